# baseline (speedup 1.0000x reference)
import functools

import jax
import jax.numpy as jnp
from jax import lax
from jax.experimental import pallas as pl
from jax.experimental.pallas import tpu as pltpu
from jax.experimental.pallas import tpu_sc as plsc

NUM_EMB = 100000
EMB_DIM = 64
HN = 3200000
ROWS = HN // EMB_DIM
BATCH = 4096
BAG = 20

NW = 32
BAGS_PER_W = BATCH // NW
CB = 4
CHUNK_IDX = CB * BAG
CHUNKS = BAGS_PER_W // CB
LANES = 16
COLS = EMB_DIM // LANES
NBUF = 2
PACK = EMB_DIM // 2


def _bag_kernel(w2, idx):
    mesh = plsc.VectorSubcoreMesh(core_axis_name="c", subcore_axis_name="s")

    @functools.partial(
        pl.kernel,
        mesh=mesh,
        compiler_params=pltpu.CompilerParams(use_tc_tiling_on_sc=False),
        out_type=jax.ShapeDtypeStruct((BATCH, EMB_DIM), jnp.float32),
        scratch_types=[
            pltpu.VMEM((CHUNKS, CHUNK_IDX), jnp.int32),
        ] + [pltpu.VMEM((CHUNK_IDX, PACK), jnp.int32)] * NBUF + [
            pltpu.VMEM((BAGS_PER_W, EMB_DIM), jnp.float32),
        ] + [pltpu.SemaphoreType.DMA] * NBUF,
    )
    def k(w2_hbm, idx_hbm, out_hbm, idx_v, *rest):
        rows = rest[:NBUF]
        out_v = rest[NBUF]
        sems = rest[NBUF + 1:]
        wid = lax.axis_index("s") * 2 + lax.axis_index("c")
        pltpu.sync_copy(idx_hbm.at[wid], idx_v)

        @pl.loop(0, CHUNKS)
        def _(c):
            for k5 in range(CHUNK_IDX // LANES):
                sl = pl.ds(k5 * LANES, LANES)
                v = idx_v[c, sl]
                idx_v[c, sl] = jnp.where(v >= ROWS, v - ROWS, v)

        def start(c, buf, sem):
            pltpu.async_copy(w2_hbm.at[idx_v.at[c]], buf, sem)

        def wait(c, buf, sem):
            pltpu.make_async_copy(w2_hbm.at[idx_v.at[c]], buf, sem).wait()

        himask = jnp.int32(-65536)

        def accum(c, buf):
            for b in range(CB):
                for g in range(2):
                    sl = pl.ds(g * LANES, LANES)
                    accs = [None, None]
                    for r in range(BAG):
                        v = buf[b * BAG + r, sl]
                        lo = lax.bitcast_convert_type(v << 16, jnp.float32)
                        hi = lax.bitcast_convert_type(v & himask, jnp.float32)
                        accs[0] = lo if accs[0] is None else accs[0] + lo
                        accs[1] = hi if accs[1] is None else accs[1] + hi
                    out_v[c * CB + b, pl.ds(g * 2 * LANES, LANES)] = accs[0]
                    out_v[c * CB + b, pl.ds((g * 2 + 1) * LANES, LANES)] = accs[1]

        for j in range(NBUF - 1):
            start(j, rows[j], sems[j])

        @pl.loop(0, CHUNKS - NBUF, step=NBUF)
        def _(c):
            for j in range(NBUF):
                start(c + j + NBUF - 1, rows[(j - 1) % NBUF], sems[(j - 1) % NBUF])
                wait(c + j, rows[j], sems[j])
                accum(c + j, rows[j])

        cl = CHUNKS - NBUF
        start(CHUNKS - 1, rows[(CHUNKS - 1) % NBUF], sems[(CHUNKS - 1) % NBUF])
        for j in range(NBUF):
            wait(cl + j, rows[j], sems[j])
            accum(cl + j, rows[j])

        pltpu.sync_copy(out_v, out_hbm.at[pl.ds(wid * BAGS_PER_W, BAGS_PER_W)])

    return k(w2, idx)


def kernel(x, hashed_weight):
    wb = hashed_weight.astype(jnp.bfloat16).reshape(ROWS, EMB_DIM)
    b16 = jax.lax.bitcast_convert_type(wb, jnp.uint16)
    g = b16.reshape(ROWS, 2, 2, LANES)
    packed = g[:, :, 0, :].astype(jnp.uint32) | (
        g[:, :, 1, :].astype(jnp.uint32) << 16)
    tbl = jax.lax.bitcast_convert_type(packed.reshape(ROWS, PACK), jnp.int32)
    idx = x.reshape(NW, CHUNKS, CHUNK_IDX)
    return _bag_kernel(tbl, idx)


# speedup vs baseline: 7.7368x; 7.7368x over previous
import functools

import jax
import jax.numpy as jnp
from jax import lax
from jax.experimental import pallas as pl
from jax.experimental.pallas import tpu as pltpu
from jax.experimental.pallas import tpu_sc as plsc

NUM_EMB = 100000
EMB_DIM = 64
HN = 3200000
ROWS = HN // EMB_DIM
BATCH = 4096
BAG = 20

NW = 32
BAGS_PER_W = BATCH // NW
CB = 4
CHUNK_IDX = CB * BAG
CHUNKS = BAGS_PER_W // CB
LANES = 16
COLS = EMB_DIM // LANES
NBUF = 2
PACK = EMB_DIM // 2


def _bag_kernel(w2, idx):
    mesh = plsc.VectorSubcoreMesh(core_axis_name="c", subcore_axis_name="s")

    @functools.partial(
        pl.kernel,
        mesh=mesh,
        compiler_params=pltpu.CompilerParams(use_tc_tiling_on_sc=False),
        out_type=jax.ShapeDtypeStruct((BATCH, EMB_DIM), jnp.bfloat16),
        scratch_types=[
            pltpu.VMEM((CHUNKS, CHUNK_IDX), jnp.int32),
        ] + [pltpu.VMEM((CHUNK_IDX, EMB_DIM), jnp.bfloat16)] * NBUF + [
            pltpu.VMEM((BAGS_PER_W, EMB_DIM), jnp.bfloat16),
        ] + [pltpu.SemaphoreType.DMA] * NBUF,
    )
    def k(w2_hbm, idx_hbm, out_hbm, idx_v, *rest):
        rows = rest[:NBUF]
        out_v = rest[NBUF]
        sems = rest[NBUF + 1:]
        wid = lax.axis_index("s") * 2 + lax.axis_index("c")
        pltpu.sync_copy(idx_hbm.at[wid], idx_v)

        @pl.loop(0, CHUNKS)
        def _(c):
            for k5 in range(CHUNK_IDX // LANES):
                sl = pl.ds(k5 * LANES, LANES)
                v = idx_v[c, sl]
                idx_v[c, sl] = jnp.where(v >= ROWS, v - ROWS, v)

        def start(c, buf, sem):
            pltpu.async_copy(w2_hbm.at[idx_v.at[c]], buf, sem)

        def wait(c, buf, sem):
            pltpu.make_async_copy(w2_hbm.at[idx_v.at[c]], buf, sem).wait()

        def accum(c, buf):
            for b in range(CB):
                for g in range(2):
                    sl = pl.ds(g * 2 * LANES, 2 * LANES)
                    vals = [buf[b * BAG + r, sl] for r in range(BAG)]
                    while len(vals) > 1:
                        nxt = [vals[i] + vals[i + 1]
                               for i in range(0, len(vals) - 1, 2)]
                        if len(vals) % 2:
                            nxt.append(vals[-1])
                        vals = nxt
                    out_v[c * CB + b, sl] = vals[0]

        for j in range(NBUF - 1):
            start(j, rows[j], sems[j])

        @pl.loop(0, CHUNKS - NBUF, step=NBUF)
        def _(c):
            for j in range(NBUF):
                start(c + j + NBUF - 1, rows[(j - 1) % NBUF], sems[(j - 1) % NBUF])
                wait(c + j, rows[j], sems[j])
                accum(c + j, rows[j])

        cl = CHUNKS - NBUF
        start(CHUNKS - 1, rows[(CHUNKS - 1) % NBUF], sems[(CHUNKS - 1) % NBUF])
        for j in range(NBUF):
            wait(cl + j, rows[j], sems[j])
            accum(cl + j, rows[j])

        pltpu.sync_copy(out_v, out_hbm.at[pl.ds(wid * BAGS_PER_W, BAGS_PER_W)])

    return k(w2, idx)


def kernel(x, hashed_weight):
    wb = hashed_weight.astype(jnp.bfloat16).reshape(ROWS, EMB_DIM)
    idx = x.reshape(NW, CHUNKS, CHUNK_IDX)
    return _bag_kernel(wb, idx).astype(jnp.float32)


# speedup vs baseline: 13.5823x; 1.7555x over previous
import functools

import jax
import jax.numpy as jnp
from jax import lax
from jax.experimental import pallas as pl
from jax.experimental.pallas import tpu as pltpu
from jax.experimental.pallas import tpu_sc as plsc

NUM_EMB = 100000
EMB_DIM = 64
HN = 3200000
ROWS = HN // EMB_DIM
BATCH = 4096
BAG = 20

NW = 32
BAGS_PER_W = BATCH // NW
CB = 4
CHUNK_IDX = CB * BAG
CHUNKS = BAGS_PER_W // CB
LANES = 16
COLS = EMB_DIM // LANES
NBUF = 2
PACK = EMB_DIM // 2


def _bag_kernel(w2, idx):
    mesh = plsc.VectorSubcoreMesh(core_axis_name="c", subcore_axis_name="s")

    @functools.partial(
        pl.kernel,
        mesh=mesh,
        compiler_params=pltpu.CompilerParams(use_tc_tiling_on_sc=False),
        out_type=jax.ShapeDtypeStruct((BATCH, EMB_DIM), jnp.float32),
        scratch_types=[
            pltpu.VMEM((CHUNKS, CHUNK_IDX), jnp.int32),
        ] + [pltpu.VMEM((CHUNK_IDX, EMB_DIM), jnp.float32)] * NBUF + [
            pltpu.VMEM((BAGS_PER_W, EMB_DIM), jnp.float32),
        ] + [pltpu.SemaphoreType.DMA] * NBUF,
    )
    def k(w2_hbm, idx_hbm, out_hbm, idx_v, *rest):
        rows = rest[:NBUF]
        out_v = rest[NBUF]
        sems = rest[NBUF + 1:]
        wid = lax.axis_index("s") * 2 + lax.axis_index("c")
        pltpu.sync_copy(idx_hbm.at[wid], idx_v)

        @pl.loop(0, CHUNKS)
        def _(c):
            for k5 in range(CHUNK_IDX // LANES):
                sl = pl.ds(k5 * LANES, LANES)
                v = idx_v[c, sl]
                idx_v[c, sl] = jnp.where(v >= ROWS, v - ROWS, v)

        def start(c, buf, sem):
            pltpu.async_copy(w2_hbm.at[idx_v.at[c]], buf, sem)

        def wait(c, buf, sem):
            pltpu.make_async_copy(w2_hbm.at[idx_v.at[c]], buf, sem).wait()

        def accum(c, buf):
            for b in range(CB):
                for g in range(COLS):
                    sl = pl.ds(g * LANES, LANES)
                    vals = [buf[b * BAG + r, sl] for r in range(BAG)]
                    while len(vals) > 1:
                        nxt = [vals[i] + vals[i + 1]
                               for i in range(0, len(vals) - 1, 2)]
                        if len(vals) % 2:
                            nxt.append(vals[-1])
                        vals = nxt
                    out_v[c * CB + b, sl] = vals[0]

        for j in range(NBUF - 1):
            start(j, rows[j], sems[j])

        @pl.loop(0, CHUNKS - NBUF, step=NBUF)
        def _(c):
            for j in range(NBUF):
                start(c + j + NBUF - 1, rows[(j - 1) % NBUF], sems[(j - 1) % NBUF])
                wait(c + j, rows[j], sems[j])
                accum(c + j, rows[j])

        cl = CHUNKS - NBUF
        start(CHUNKS - 1, rows[(CHUNKS - 1) % NBUF], sems[(CHUNKS - 1) % NBUF])
        for j in range(NBUF):
            wait(cl + j, rows[j], sems[j])
            accum(cl + j, rows[j])

        pltpu.sync_copy(out_v, out_hbm.at[pl.ds(wid * BAGS_PER_W, BAGS_PER_W)])

    return k(w2, idx)


def kernel(x, hashed_weight):
    w2 = hashed_weight.reshape(ROWS, EMB_DIM)
    idx = x.reshape(NW, CHUNKS, CHUNK_IDX)
    return _bag_kernel(w2, idx)


# speedup vs baseline: 13.6085x; 1.0019x over previous
import functools

import jax
import jax.numpy as jnp
from jax import lax
from jax.experimental import pallas as pl
from jax.experimental.pallas import tpu as pltpu
from jax.experimental.pallas import tpu_sc as plsc

NUM_EMB = 100000
EMB_DIM = 64
HN = 3200000
ROWS = HN // EMB_DIM
BATCH = 4096
BAG = 20

NW = 32
BAGS_PER_W = BATCH // NW
CB = 4
CHUNK_IDX = CB * BAG
CHUNKS = BAGS_PER_W // CB
LANES = 16
COLS = EMB_DIM // LANES
NBUF = 2
PACK = EMB_DIM // 2


def _bag_kernel(w2, idx):
    mesh = plsc.VectorSubcoreMesh(core_axis_name="c", subcore_axis_name="s")

    @functools.partial(
        pl.kernel,
        mesh=mesh,
        compiler_params=pltpu.CompilerParams(
            use_tc_tiling_on_sc=False,
            skip_device_barrier=True,
            disable_bounds_checks=True,
            disable_semaphore_checks=True,
        ),
        out_type=jax.ShapeDtypeStruct((BATCH, EMB_DIM), jnp.float32),
        scratch_types=[
            pltpu.VMEM((CHUNKS, CHUNK_IDX), jnp.int32),
        ] + [pltpu.VMEM((CHUNK_IDX, EMB_DIM), jnp.float32)] * NBUF + [
            pltpu.VMEM((BAGS_PER_W, EMB_DIM), jnp.float32),
        ] + [pltpu.SemaphoreType.DMA] * NBUF,
    )
    def k(w2_hbm, idx_hbm, out_hbm, idx_v, *rest):
        rows = rest[:NBUF]
        out_v = rest[NBUF]
        sems = rest[NBUF + 1:]
        wid = lax.axis_index("s") * 2 + lax.axis_index("c")
        pltpu.sync_copy(idx_hbm.at[wid], idx_v)

        @pl.loop(0, CHUNKS)
        def _(c):
            for k5 in range(CHUNK_IDX // LANES):
                sl = pl.ds(k5 * LANES, LANES)
                v = idx_v[c, sl]
                idx_v[c, sl] = jnp.where(v >= ROWS, v - ROWS, v)

        def start(c, buf, sem):
            pltpu.async_copy(w2_hbm.at[idx_v.at[c]], buf, sem)

        def wait(c, buf, sem):
            pltpu.make_async_copy(w2_hbm.at[idx_v.at[c]], buf, sem).wait()

        def accum(c, buf):
            for b in range(CB):
                for g in range(COLS):
                    sl = pl.ds(g * LANES, LANES)
                    vals = [buf[b * BAG + r, sl] for r in range(BAG)]
                    while len(vals) > 1:
                        nxt = [vals[i] + vals[i + 1]
                               for i in range(0, len(vals) - 1, 2)]
                        if len(vals) % 2:
                            nxt.append(vals[-1])
                        vals = nxt
                    out_v[c * CB + b, sl] = vals[0]

        for j in range(NBUF - 1):
            start(j, rows[j], sems[j])

        @pl.loop(0, CHUNKS - NBUF, step=NBUF)
        def _(c):
            for j in range(NBUF):
                start(c + j + NBUF - 1, rows[(j - 1) % NBUF], sems[(j - 1) % NBUF])
                wait(c + j, rows[j], sems[j])
                accum(c + j, rows[j])

        cl = CHUNKS - NBUF
        start(CHUNKS - 1, rows[(CHUNKS - 1) % NBUF], sems[(CHUNKS - 1) % NBUF])
        for j in range(NBUF):
            wait(cl + j, rows[j], sems[j])
            accum(cl + j, rows[j])

        pltpu.sync_copy(out_v, out_hbm.at[pl.ds(wid * BAGS_PER_W, BAGS_PER_W)])

    return k(w2, idx)


def kernel(x, hashed_weight):
    w2 = hashed_weight.reshape(ROWS, EMB_DIM)
    idx = x.reshape(NW, CHUNKS, CHUNK_IDX)
    return _bag_kernel(w2, idx)
